# baseline (speedup 1.0000x reference)
import functools

import jax
import jax.numpy as jnp
from jax import lax
from jax.experimental import pallas as pl
from jax.experimental.pallas import tpu as pltpu
from jax.experimental.pallas import tpu_sc as plsc

N = 10000
DEG = 32
D = 128
BN_EPS = 1e-5

CHUNK = 8
ROWS = CHUNK * DEG
NCH = N // CHUNK
NSC = 16
NCH_H = NCH // 2
PW = NCH_H // NSC
EXTRA = NCH_H - PW * NSC
LANES = 16
D2 = D // 2
NSEG = D2 // LANES


def _make_pool_body(ch0):
    def _pool_body(x_hbm, nbr_hbm, out_hbm, idx_all, rows_v, pool_v,
                   sem0, sem1, stsem0, stsem1):
        wid = lax.axis_index("s")
        base_ch = ch0 + wid * PW

        pltpu.sync_copy(nbr_hbm.at[pl.ds(base_ch, PW)],
                        idx_all.at[pl.ds(0, PW)])

        @pl.when(wid < EXTRA)
        def _():
            pltpu.sync_copy(nbr_hbm.at[pl.ds(ch0 + NSC * PW + wid, 1)],
                            idx_all.at[pl.ds(PW, 1)])

        sems = (sem0, sem1)

        def start(l, b):
            for g in range(2):
                pltpu.async_copy(
                    x_hbm.at[idx_all.at[l, g]],
                    rows_v.at[b, pl.ds(g * 128, 128)], sems[b])

        def wait(l, b):
            for g in range(2):
                pltpu.make_async_copy(
                    x_hbm.at[idx_all.at[l, g]],
                    rows_v.at[b, pl.ds(g * 128, 128)], sems[b]).wait()

        stsems = (stsem0, stsem1)
        lo_mask = jnp.int32(0xFFFF)

        def compute(ch, b):
            def node_body(k, carry):
                base = k * DEG
                w0 = [rows_v[b, base, pl.ds(d * LANES, LANES)]
                      for d in range(NSEG)]
                los = [w & lo_mask for w in w0]
                his = [lax.shift_right_logical(w, 16) for w in w0]
                for j in range(1, DEG):
                    for d in range(NSEG):
                        w = rows_v[b, base + j, pl.ds(d * LANES, LANES)]
                        los[d] = jnp.maximum(los[d], w & lo_mask)
                        his[d] = jnp.maximum(
                            his[d], lax.shift_right_logical(w, 16))
                for d in range(NSEG):
                    lo, hi = los[d], his[d]
                    lo = lo ^ (lo_mask ^ (lax.shift_right_logical(lo, 15)
                                          * jnp.int32(0x7FFF)))
                    hi = hi ^ (lo_mask ^ (lax.shift_right_logical(hi, 15)
                                          * jnp.int32(0x7FFF)))
                    pool_v[b, k, pl.ds(d * LANES, LANES)] = (
                        lax.shift_left(hi, 16) | lo)
                return carry

            lax.fori_loop(0, CHUNK, node_body, 0)
            pltpu.async_copy(
                pool_v.at[b],
                out_hbm.at[pl.ds((ch - ch0) * CHUNK, CHUNK)], stsems[b])

        def wait_store(b):
            pltpu.make_async_copy(pool_v.at[b], out_hbm.at[pl.ds(0, CHUNK)],
                                  stsems[b]).wait()

        start(0, 0)

        def pair_body(g, carry):
            l = g * 2
            start(l + 1, 1)
            wait(l, 0)

            @pl.when(g > 0)
            def _():
                wait_store(0)

            compute(base_ch + l, 0)
            start(l + 2, 0)
            wait(l + 1, 1)

            @pl.when(g > 0)
            def _():
                wait_store(1)

            compute(base_ch + l + 1, 1)
            return carry

        lax.fori_loop(0, PW // 2, pair_body, 0)

        @pl.when(wid < EXTRA)
        def _():
            start(PW, 1)

        wait(PW - 1, 0)
        wait_store(0)
        compute(base_ch + PW - 1, 0)

        @pl.when(wid < EXTRA)
        def _():
            wait(PW, 1)
            wait_store(1)
            compute(ch0 + NSC * PW + wid, 1)

        wait_store(0)
        wait_store(1)

    return _pool_body


def _make_pool(ch0):
    return functools.partial(
        pl.kernel,
        out_type=jax.ShapeDtypeStruct((NCH_H * CHUNK, D2), jnp.int32),
        mesh=plsc.VectorSubcoreMesh(core_axis_name="c", subcore_axis_name="s",
                                    num_cores=1),
        compiler_params=pltpu.CompilerParams(use_tc_tiling_on_sc=False),
        scratch_types=[
            pltpu.VMEM((PW + 1, 2, 128), jnp.int32),
            pltpu.VMEM((2, ROWS, D2), jnp.int32),
            pltpu.VMEM((2, CHUNK, D2), jnp.int32),
            pltpu.SemaphoreType.DMA,
            pltpu.SemaphoreType.DMA,
            pltpu.SemaphoreType.DMA,
            pltpu.SemaphoreType.DMA,
        ],
    )(_make_pool_body(ch0))


_pool_a = _make_pool(0)
_pool_b = _make_pool(NCH_H)


def _enc_body(x_ref, out_ref):
    xv = x_ref[...]

    def code(xs):
        v = jax.lax.bitcast_convert_type(xs, jnp.int32)
        s = lax.shift_right_logical(v, 31)
        e = lax.shift_right_logical(v, 23) & jnp.int32(0xFF)
        biased = (v & jnp.int32(0x7FFFFFFF)) - jnp.int32(112 << 23)
        lsb = lax.shift_right_logical(biased, 13) & jnp.int32(1)
        hmag = lax.shift_right_logical(biased + jnp.int32(0xFFF) + lsb, 13)
        hmag = jnp.minimum(hmag, jnp.int32(0x7BFF))
        hmag = jnp.where(e < 113, jnp.int32(0), hmag)
        return jnp.where(s == 0, hmag | jnp.int32(0x8000),
                         jnp.int32(0x7FFF) - hmag)

    out_ref[...] = code(xv[:, :D2]) | lax.shift_left(code(xv[:, D2:]), 16)


def _mlp_body(pa_ref, pb_ref, x_ref, w1_ref, b1_ref, g1_ref, be1_ref,
              w2_ref, b2_ref, g2_ref, be2_ref, out_ref):
    def dec(hs):
        s = lax.shift_left(lax.shift_right_logical(hs, 15), 31)
        hm = hs & jnp.int32(0x7FFF)
        bits = jnp.where(hm == 0, s,
                         s | (lax.shift_left(hm, 13) + jnp.int32(112 << 23)))
        return jax.lax.bitcast_convert_type(bits, jnp.float32)

    w = jnp.concatenate([pa_ref[...], pb_ref[...]], axis=0)
    pooled = jnp.concatenate(
        [dec(w & jnp.int32(0xFFFF)), dec(lax.shift_right_logical(w, 16))],
        axis=1)
    h = pooled + x_ref[...]
    h = jnp.dot(h, w1_ref[...], preferred_element_type=jnp.float32) + b1_ref[...]
    mean = jnp.mean(h, axis=0, keepdims=True)
    ctr = h - mean
    var = jnp.mean(ctr * ctr, axis=0, keepdims=True)
    h = ctr * lax.rsqrt(var + BN_EPS) * g1_ref[...] + be1_ref[...]
    h = jnp.maximum(h, 0.0)
    h = jnp.dot(h, w2_ref[...], preferred_element_type=jnp.float32) + b2_ref[...]
    mean = jnp.mean(h, axis=0, keepdims=True)
    ctr = h - mean
    var = jnp.mean(ctr * ctr, axis=0, keepdims=True)
    h = ctr * lax.rsqrt(var + BN_EPS) * g2_ref[...] + be2_ref[...]
    out_ref[...] = jnp.maximum(h, 0.0)


def kernel(x, padded_neighbors, W1, b1, g1, be1, W2, b2, g2, be2):
    nbr3 = padded_neighbors.reshape(NCH, 2, 128)
    coded = pl.pallas_call(
        _enc_body,
        out_shape=jax.ShapeDtypeStruct((N, D2), jnp.int32),
    )(x)
    pa = _pool_a(coded, nbr3)
    pb = _pool_b(coded, nbr3)
    out = pl.pallas_call(
        _mlp_body,
        out_shape=jax.ShapeDtypeStruct((N, D), jnp.float32),
    )(pa, pb, x, W1.T, b1.reshape(1, D), g1.reshape(1, D),
      be1.reshape(1, D), W2.T, b2.reshape(1, D), g2.reshape(1, D),
      be2.reshape(1, D))
    return out


# speedup vs baseline: 1.4466x; 1.4466x over previous
import functools

import jax
import jax.numpy as jnp
from jax import lax
from jax.experimental import pallas as pl
from jax.experimental.pallas import tpu as pltpu
from jax.experimental.pallas import tpu_sc as plsc

N = 10000
DEG = 32
D = 128
BN_EPS = 1e-5

CHUNK = 8
ROWS = CHUNK * DEG
NCH = N // CHUNK
NW = 32
PW = NCH // NW
EXTRA = NCH - PW * NW
LANES = 16
D2 = D // 2
NSEG = D2 // LANES


def _pool_body(x_hbm, nbr_hbm, out_hbm, idx_all, rows_v, pool_v,
               sem0, sem1, stsem0, stsem1):
    wid = lax.axis_index("s") * 2 + lax.axis_index("c")
    base_ch = wid * PW

    pltpu.sync_copy(nbr_hbm.at[pl.ds(base_ch, PW)], idx_all.at[pl.ds(0, PW)])

    @pl.when(wid < EXTRA)
    def _():
        pltpu.sync_copy(nbr_hbm.at[pl.ds(NW * PW + wid, 1)],
                        idx_all.at[pl.ds(PW, 1)])

    sems = (sem0, sem1)

    def start(l, b):
        for g in range(2):
            pltpu.async_copy(
                x_hbm.at[idx_all.at[l, g]],
                rows_v.at[b, pl.ds(g * 128, 128)], sems[b])

    def wait(l, b):
        for g in range(2):
            pltpu.make_async_copy(
                x_hbm.at[idx_all.at[l, g]],
                rows_v.at[b, pl.ds(g * 128, 128)], sems[b]).wait()

    stsems = (stsem0, stsem1)

    lo_mask = jnp.int32(0xFFFF)

    def compute(ch, b):
        def node_body(k, carry):
            base = k * DEG
            w0 = [rows_v[b, base, pl.ds(d * LANES, LANES)]
                  for d in range(NSEG)]
            los = [w & lo_mask for w in w0]
            his = [lax.shift_right_logical(w, 16) for w in w0]
            for j in range(1, DEG):
                for d in range(NSEG):
                    w = rows_v[b, base + j, pl.ds(d * LANES, LANES)]
                    los[d] = jnp.maximum(los[d], w & lo_mask)
                    his[d] = jnp.maximum(his[d], lax.shift_right_logical(w, 16))
            for d in range(NSEG):
                lo, hi = los[d], his[d]
                lo = lo ^ (lo_mask ^ (lax.shift_right_logical(lo, 15) * jnp.int32(0x7FFF)))
                hi = hi ^ (lo_mask ^ (lax.shift_right_logical(hi, 15) * jnp.int32(0x7FFF)))
                pool_v[b, k, pl.ds(d * LANES, LANES)] = (
                    lax.shift_left(hi, 16) | lo)
            return carry

        lax.fori_loop(0, CHUNK, node_body, 0)
        pltpu.async_copy(pool_v.at[b], out_hbm.at[pl.ds(ch * CHUNK, CHUNK)],
                         stsems[b])

    def wait_store(b):
        pltpu.make_async_copy(pool_v.at[b], out_hbm.at[pl.ds(0, CHUNK)],
                              stsems[b]).wait()

    start(0, 0)

    def pair_body(g, carry):
        l = g * 2
        start(l + 1, 1)
        wait(l, 0)

        @pl.when(g > 0)
        def _():
            wait_store(0)

        compute(base_ch + l, 0)
        start(l + 2, 0)
        wait(l + 1, 1)

        @pl.when(g > 0)
        def _():
            wait_store(1)

        compute(base_ch + l + 1, 1)
        return carry

    lax.fori_loop(0, PW // 2, pair_body, 0)

    @pl.when(wid < EXTRA)
    def _():
        start(PW, 1)

    wait(PW - 1, 0)
    wait_store(0)
    compute(base_ch + PW - 1, 0)

    @pl.when(wid < EXTRA)
    def _():
        wait(PW, 1)
        wait_store(1)
        compute(NW * PW + wid, 1)

    wait_store(0)
    wait_store(1)


_pool = functools.partial(
    pl.kernel,
    out_type=jax.ShapeDtypeStruct((N, D2), jnp.int32),
    mesh=plsc.VectorSubcoreMesh(core_axis_name="c", subcore_axis_name="s"),
    compiler_params=pltpu.CompilerParams(use_tc_tiling_on_sc=False),
    scratch_types=[
        pltpu.VMEM((PW + 1, 2, 128), jnp.int32),
        pltpu.VMEM((2, ROWS, D2), jnp.int32),
        pltpu.VMEM((2, CHUNK, D2), jnp.int32),
        pltpu.SemaphoreType.DMA,
        pltpu.SemaphoreType.DMA,
        pltpu.SemaphoreType.DMA,
        pltpu.SemaphoreType.DMA,
    ],
)(_pool_body)


def _enc_body(x_ref, out_ref):
    xv = x_ref[...]

    def code(xs):
        v = jax.lax.bitcast_convert_type(xs, jnp.int32)
        s = lax.shift_right_logical(v, 31)
        e = lax.shift_right_logical(v, 23) & jnp.int32(0xFF)
        biased = (v & jnp.int32(0x7FFFFFFF)) - jnp.int32(112 << 23)
        lsb = lax.shift_right_logical(biased, 13) & jnp.int32(1)
        hmag = lax.shift_right_logical(biased + jnp.int32(0xFFF) + lsb, 13)
        hmag = jnp.minimum(hmag, jnp.int32(0x7BFF))
        hmag = jnp.where(e < 113, jnp.int32(0), hmag)
        return jnp.where(s == 0, hmag | jnp.int32(0x8000),
                         jnp.int32(0x7FFF) - hmag)

    out_ref[...] = code(xv[:, :D2]) | lax.shift_left(code(xv[:, D2:]), 16)


def _mlp_body(pooled_ref, x_ref, w1_ref, b1_ref, g1_ref, be1_ref,
              w2_ref, b2_ref, g2_ref, be2_ref, out_ref):
    def dec(hs):
        s = lax.shift_left(lax.shift_right_logical(hs, 15), 31)
        hm = hs & jnp.int32(0x7FFF)
        bits = jnp.where(hm == 0, s,
                         s | (lax.shift_left(hm, 13) + jnp.int32(112 << 23)))
        return jax.lax.bitcast_convert_type(bits, jnp.float32)

    w = pooled_ref[...]
    pooled = jnp.concatenate(
        [dec(w & jnp.int32(0xFFFF)), dec(lax.shift_right_logical(w, 16))],
        axis=1)
    h = pooled + x_ref[...]
    h = lax.dot_general(h, w1_ref[...], (((1,), (1,)), ((), ())),
                        preferred_element_type=jnp.float32) + b1_ref[...]
    mean = jnp.mean(h, axis=0, keepdims=True)
    ctr = h - mean
    var = jnp.mean(ctr * ctr, axis=0, keepdims=True)
    h = ctr * lax.rsqrt(var + BN_EPS) * g1_ref[...] + be1_ref[...]
    h = jnp.maximum(h, 0.0)
    h = lax.dot_general(h, w2_ref[...], (((1,), (1,)), ((), ())),
                        preferred_element_type=jnp.float32) + b2_ref[...]
    mean = jnp.mean(h, axis=0, keepdims=True)
    ctr = h - mean
    var = jnp.mean(ctr * ctr, axis=0, keepdims=True)
    h = ctr * lax.rsqrt(var + BN_EPS) * g2_ref[...] + be2_ref[...]
    out_ref[...] = jnp.maximum(h, 0.0)


def kernel(x, padded_neighbors, W1, b1, g1, be1, W2, b2, g2, be2):
    nbr3 = padded_neighbors.reshape(NCH, 2, 128)
    coded = pl.pallas_call(
        _enc_body,
        out_shape=jax.ShapeDtypeStruct((N, D2), jnp.int32),
    )(x)
    pooled_b = _pool(coded, nbr3)
    out = pl.pallas_call(
        _mlp_body,
        out_shape=jax.ShapeDtypeStruct((N, D), jnp.float32),
    )(pooled_b, x, W1, b1.reshape(1, D), g1.reshape(1, D),
      be1.reshape(1, D), W2, b2.reshape(1, D), g2.reshape(1, D),
      be2.reshape(1, D))
    return out


# speedup vs baseline: 1.5468x; 1.0693x over previous
import functools

import jax
import jax.numpy as jnp
from jax import lax
from jax.experimental import pallas as pl
from jax.experimental.pallas import tpu as pltpu
from jax.experimental.pallas import tpu_sc as plsc

N = 10000
DEG = 32
D = 128
BN_EPS = 1e-5

CHUNK = 8
ROWS = CHUNK * DEG
NCH = N // CHUNK
NW = 32
PW = NCH // NW
EXTRA = NCH - PW * NW
LANES = 16
D2 = D // 2
NSEG = D2 // LANES


def _pool_body(x_hbm, nbr_hbm, out_hbm, idx_all, rows_v, pool_v,
               sem0, sem1, sem2, stsem0, stsem1, stsem2):
    wid = lax.axis_index("s") * 2 + lax.axis_index("c")
    base_ch = wid * PW

    pltpu.sync_copy(nbr_hbm.at[pl.ds(base_ch, PW)], idx_all.at[pl.ds(0, PW)])

    @pl.when(wid < EXTRA)
    def _():
        pltpu.sync_copy(nbr_hbm.at[pl.ds(NW * PW + wid, 1)],
                        idx_all.at[pl.ds(PW, 1)])

    sems = (sem0, sem1, sem2)

    def start(l, b):
        for g in range(2):
            pltpu.async_copy(
                x_hbm.at[idx_all.at[l, g]],
                rows_v.at[b, pl.ds(g * 128, 128)], sems[b])

    def wait(l, b):
        for g in range(2):
            pltpu.make_async_copy(
                x_hbm.at[idx_all.at[l, g]],
                rows_v.at[b, pl.ds(g * 128, 128)], sems[b]).wait()

    stsems = (stsem0, stsem1, stsem2)

    lo_mask = jnp.int32(0xFFFF)

    def compute(ch, b):
        def node_body(k, carry):
            base = k * DEG
            w0 = [rows_v[b, base, pl.ds(d * LANES, LANES)]
                  for d in range(NSEG)]
            los = [w & lo_mask for w in w0]
            his = [lax.shift_right_logical(w, 16) for w in w0]
            for j in range(1, DEG):
                for d in range(NSEG):
                    w = rows_v[b, base + j, pl.ds(d * LANES, LANES)]
                    los[d] = jnp.maximum(los[d], w & lo_mask)
                    his[d] = jnp.maximum(his[d], lax.shift_right_logical(w, 16))
            for d in range(NSEG):
                lo, hi = los[d], his[d]
                lo = lo ^ (lo_mask ^ (lax.shift_right_logical(lo, 15) * jnp.int32(0x7FFF)))
                hi = hi ^ (lo_mask ^ (lax.shift_right_logical(hi, 15) * jnp.int32(0x7FFF)))
                pool_v[b, k, pl.ds(d * LANES, LANES)] = (
                    lax.shift_left(hi, 16) | lo)
            return carry

        lax.fori_loop(0, CHUNK, node_body, 0)
        pltpu.async_copy(pool_v.at[b], out_hbm.at[pl.ds(ch * CHUNK, CHUNK)],
                         stsems[b])

    def wait_store(b):
        pltpu.make_async_copy(pool_v.at[b], out_hbm.at[pl.ds(0, CHUNK)],
                              stsems[b]).wait()

    start(0, 0)
    start(1, 1)

    def trip_body(g, carry):
        l = g * 3
        start(l + 2, 2)
        wait(l, 0)

        @pl.when(g > 0)
        def _():
            wait_store(0)

        compute(base_ch + l, 0)

        @pl.when(l + 3 < PW)
        def _():
            start(l + 3, 0)

        wait(l + 1, 1)

        @pl.when(g > 0)
        def _():
            wait_store(1)

        compute(base_ch + l + 1, 1)

        @pl.when(l + 4 < PW)
        def _():
            start(l + 4, 1)

        wait(l + 2, 2)

        @pl.when(g > 0)
        def _():
            wait_store(2)

        compute(base_ch + l + 2, 2)
        return carry

    lax.fori_loop(0, PW // 3, trip_body, 0)

    @pl.when(wid < EXTRA)
    def _():
        start(PW, 0)
        wait(PW, 0)
        wait_store(0)
        compute(NW * PW + wid, 0)

    wait_store(0)
    wait_store(1)
    wait_store(2)


_pool = functools.partial(
    pl.kernel,
    out_type=jax.ShapeDtypeStruct((N, D2), jnp.int32),
    mesh=plsc.VectorSubcoreMesh(core_axis_name="c", subcore_axis_name="s"),
    compiler_params=pltpu.CompilerParams(use_tc_tiling_on_sc=False),
    scratch_types=[
        pltpu.VMEM((PW + 1, 2, 128), jnp.int32),
        pltpu.VMEM((3, ROWS, D2), jnp.int32),
        pltpu.VMEM((3, CHUNK, D2), jnp.int32),
        pltpu.SemaphoreType.DMA,
        pltpu.SemaphoreType.DMA,
        pltpu.SemaphoreType.DMA,
        pltpu.SemaphoreType.DMA,
        pltpu.SemaphoreType.DMA,
        pltpu.SemaphoreType.DMA,
    ],
)(_pool_body)


def _enc_body(x_ref, out_ref):
    xv = x_ref[...]

    def code(xs):
        v = jax.lax.bitcast_convert_type(xs, jnp.int32)
        s = lax.shift_right_logical(v, 31)
        e = lax.shift_right_logical(v, 23) & jnp.int32(0xFF)
        biased = (v & jnp.int32(0x7FFFFFFF)) - jnp.int32(112 << 23)
        lsb = lax.shift_right_logical(biased, 13) & jnp.int32(1)
        hmag = lax.shift_right_logical(biased + jnp.int32(0xFFF) + lsb, 13)
        hmag = jnp.minimum(hmag, jnp.int32(0x7BFF))
        hmag = jnp.where(e < 113, jnp.int32(0), hmag)
        return jnp.where(s == 0, hmag | jnp.int32(0x8000),
                         jnp.int32(0x7FFF) - hmag)

    out_ref[...] = code(xv[:, :D2]) | lax.shift_left(code(xv[:, D2:]), 16)


def _mlp_body(pooled_ref, x_ref, w1_ref, b1_ref, g1_ref, be1_ref,
              w2_ref, b2_ref, g2_ref, be2_ref, out_ref):
    def dec(hs):
        s = lax.shift_left(lax.shift_right_logical(hs, 15), 31)
        hm = hs & jnp.int32(0x7FFF)
        bits = jnp.where(hm == 0, s,
                         s | (lax.shift_left(hm, 13) + jnp.int32(112 << 23)))
        return jax.lax.bitcast_convert_type(bits, jnp.float32)

    w = pooled_ref[...]
    pooled = jnp.concatenate(
        [dec(w & jnp.int32(0xFFFF)), dec(lax.shift_right_logical(w, 16))],
        axis=1)
    h = pooled + x_ref[...]
    h = lax.dot_general(h, w1_ref[...], (((1,), (1,)), ((), ())),
                        preferred_element_type=jnp.float32) + b1_ref[...]
    mean = jnp.mean(h, axis=0, keepdims=True)
    ctr = h - mean
    var = jnp.mean(ctr * ctr, axis=0, keepdims=True)
    h = ctr * lax.rsqrt(var + BN_EPS) * g1_ref[...] + be1_ref[...]
    h = jnp.maximum(h, 0.0)
    h = lax.dot_general(h, w2_ref[...], (((1,), (1,)), ((), ())),
                        preferred_element_type=jnp.float32) + b2_ref[...]
    mean = jnp.mean(h, axis=0, keepdims=True)
    ctr = h - mean
    var = jnp.mean(ctr * ctr, axis=0, keepdims=True)
    h = ctr * lax.rsqrt(var + BN_EPS) * g2_ref[...] + be2_ref[...]
    out_ref[...] = jnp.maximum(h, 0.0)


def kernel(x, padded_neighbors, W1, b1, g1, be1, W2, b2, g2, be2):
    nbr3 = padded_neighbors.reshape(NCH, 2, 128)
    coded = pl.pallas_call(
        _enc_body,
        out_shape=jax.ShapeDtypeStruct((N, D2), jnp.int32),
    )(x)
    pooled_b = _pool(coded, nbr3)
    out = pl.pallas_call(
        _mlp_body,
        out_shape=jax.ShapeDtypeStruct((N, D), jnp.float32),
    )(pooled_b, x, W1, b1.reshape(1, D), g1.reshape(1, D),
      be1.reshape(1, D), W2, b2.reshape(1, D), g2.reshape(1, D),
      be2.reshape(1, D))
    return out
